# Initial kernel scaffold; baseline (speedup 1.0000x reference)
import jax
import jax.numpy as jnp
from jax.experimental import pallas as pl


def kernel(h, x, v, params, senders, receivers):
    raise NotImplementedError("write your pallas kernel here")


# speedup vs baseline: 8.0391x; 8.0391x over previous
import functools

import jax
import jax.numpy as jnp
from jax import lax
from jax.experimental import pallas as pl
from jax.experimental.pallas import tpu as pltpu
from jax.experimental.pallas import tpu_sc as plsc

N = 10000
E = 160000
F32 = jnp.float32

_MESH = dict(core_axis_name="c", subcore_axis_name="s")
NC, NS = 2, 16
NW = NC * NS


def _sc_gather_nodes(htab, xtab, sidx, ridx):
    EPW = E // NW
    CH = 200

    @functools.partial(
        pl.kernel,
        mesh=plsc.VectorSubcoreMesh(**_MESH),
        out_type=[
            jax.ShapeDtypeStruct((E, 128), F32),
            jax.ShapeDtypeStruct((E, 128), F32),
            jax.ShapeDtypeStruct((E, 128), F32),
            jax.ShapeDtypeStruct((E, 128), F32),
        ],
        scratch_types=[
            pltpu.VMEM((EPW,), jnp.int32),
            pltpu.VMEM((CH, 128), F32),
            pltpu.VMEM((CH, 128), F32),
            pltpu.SemaphoreType.DMA,
        ],
    )
    def k(htab_h, xtab_h, sidx_h, ridx_h, hs_o, hr_o, xs_o, xr_o,
          idx_v, hbuf, xbuf, sem):
        wid = lax.axis_index("s") * NC + lax.axis_index("c")
        base = wid * EPW
        for idx_h, h_o, x_o in ((sidx_h, hs_o, xs_o), (ridx_h, hr_o, xr_o)):
            pltpu.sync_copy(idx_h.at[pl.ds(base, EPW)], idx_v)

            def body(i, _, h_o=h_o, x_o=x_o):
                off = i * CH
                ids = idx_v.at[pl.ds(off, CH)]
                pltpu.async_copy(htab_h.at[ids], hbuf, sem).wait()
                pltpu.sync_copy(hbuf, h_o.at[pl.ds(base + off, CH)])
                pltpu.async_copy(xtab_h.at[ids], xbuf, sem).wait()
                pltpu.sync_copy(xbuf, x_o.at[pl.ds(base + off, CH)])
                return 0

            lax.fori_loop(0, EPW // CH, body, 0)

    return k(htab, xtab, sidx, ridx)


def _sc_gather_dcnt(dcnt, ridx):
    EPW = E // NW
    CH = 200

    @functools.partial(
        pl.kernel,
        mesh=plsc.VectorSubcoreMesh(**_MESH),
        out_type=jax.ShapeDtypeStruct((E, 128), F32),
        scratch_types=[
            pltpu.VMEM((EPW,), jnp.int32),
            pltpu.VMEM((CH, 128), F32),
            pltpu.SemaphoreType.DMA,
        ],
    )
    def k(dcnt_h, ridx_h, out_o, idx_v, buf, sem):
        wid = lax.axis_index("s") * NC + lax.axis_index("c")
        base = wid * EPW
        pltpu.sync_copy(ridx_h.at[pl.ds(base, EPW)], idx_v)

        def body(i, _):
            off = i * CH
            pltpu.async_copy(dcnt_h.at[idx_v.at[pl.ds(off, CH)]], buf, sem).wait()
            pltpu.sync_copy(buf, out_o.at[pl.ds(base + off, CH)])
            return 0

        lax.fori_loop(0, EPW // CH, body, 0)

    return k(dcnt, ridx)


HN = N // 2
HNP = HN + 16


def _remap(idx_v, idx2_v, ch, lo):
    ng = ch // 16
    offs = [g * 16 for g in range(ng)]
    if ch % 16:
        offs.append(ch - 16)

    for off in offs:
        v = idx_v[pl.ds(off, 16)]
        rel = v - lo
        inb = jnp.logical_and(rel >= 0, rel < HN)
        idx2_v[pl.ds(off, 16)] = jnp.where(inb, rel, HN)


def _sc_scatter_dcnt(a16, ridx, zeros):
    EPT = E // NS
    CH = 400
    NCHK = EPT // CH

    @functools.partial(
        pl.kernel,
        mesh=plsc.VectorSubcoreMesh(**_MESH),
        out_type=jax.ShapeDtypeStruct((N, 128), F32),
        scratch_types=[
            pltpu.VMEM((CH,), jnp.int32),
            pltpu.VMEM((CH,), jnp.int32),
            pltpu.VMEM((CH, 128), F32),
            pltpu.VMEM_SHARED((HNP, 128), F32),
        ],
    )
    def k(a16_h, ridx_h, zero_h, out_o, idx_v, idx2_v, buf, acc):
        cid = lax.axis_index("c")
        sid = lax.axis_index("s")
        base = sid * EPT
        lo = cid * HN

        @pl.when(sid == 0)
        def _():
            pltpu.sync_copy(zero_h.at[pl.ds(0, HNP)], acc)

        plsc.subcore_barrier()

        def body(j, _):
            pltpu.sync_copy(ridx_h.at[pl.ds(base + j * CH, CH)], idx_v)
            _remap(idx_v, idx2_v, CH, lo)
            pltpu.sync_copy(a16_h.at[pl.ds(base + j * CH, CH)], buf)
            pltpu.sync_copy(buf, acc.at[idx2_v], add=True)
            return 0

        lax.fori_loop(0, NCHK, body, 0)
        plsc.subcore_barrier()

        @pl.when(sid == 0)
        def _():
            pltpu.sync_copy(acc.at[pl.ds(0, HN)], out_o.at[pl.ds(lo, HN)])

    return k(a16, ridx, zeros)


DSH = 128
NSH = 10
PW = NSH * DSH


def _sc_scatter_big(p, ridx, zeros):
    EPT = E // NS
    CH = 400
    NCHK = EPT // CH

    @functools.partial(
        pl.kernel,
        mesh=plsc.VectorSubcoreMesh(**_MESH),
        out_type=jax.ShapeDtypeStruct((N, PW), F32),
        scratch_types=[
            pltpu.VMEM((CH,), jnp.int32),
            pltpu.VMEM((CH,), jnp.int32),
            pltpu.VMEM((CH, DSH), F32),
            pltpu.VMEM_SHARED((HNP, DSH), F32),
        ],
    )
    def k(p_h, ridx_h, zero_h, out_o, idx_v, idx2_v, buf, acc):
        cid = lax.axis_index("c")
        sid = lax.axis_index("s")
        base = sid * EPT

        for i in range(NSH):
            lo = (i // 5) * HN
            coff = ((i % 5) * NC + cid) * DSH

            @pl.when(sid == 0)
            def _():
                pltpu.sync_copy(zero_h.at[pl.ds(0, HNP)], acc)

            plsc.subcore_barrier()

            def body(j, _, coff=coff, lo=lo):
                pltpu.sync_copy(ridx_h.at[pl.ds(base + j * CH, CH)], idx_v)
                _remap(idx_v, idx2_v, CH, lo)
                pltpu.sync_copy(
                    p_h.at[pl.ds(base + j * CH, CH), pl.ds(coff, DSH)], buf)
                pltpu.sync_copy(buf, acc.at[idx2_v], add=True)
                return 0

            lax.fori_loop(0, NCHK, body, 0)
            plsc.subcore_barrier()

            @pl.when(sid == 0)
            def _():
                pltpu.sync_copy(acc.at[pl.ds(0, HN)],
                                out_o.at[pl.ds(lo, HN), pl.ds(coff, DSH)])

            plsc.subcore_barrier()

    return k(p, ridx, zeros)


def _silu(z):
    return z * jax.nn.sigmoid(z)


def _edge_a_body(hs, hr, xs, xr, wins, winr, bin_, betas, means,
                 w1hs, w1hr, w1x, w1n, b1, w2, b2, watt, batt,
                 mtx_o, a16_o, xu8_o):
    hs_, hr_ = hs[...], hr[...]
    dx = xs[:, 0:3] - xr[:, 0:3]
    n2 = jnp.sum(dx * dx, axis=1, keepdims=True)
    norm = jnp.sqrt(jnp.maximum(n2, 0.0) + 1e-10)
    h_in = hs_ @ wins[...] + hr_ @ winr[...] + bin_[...]
    rbf = jnp.exp(-betas[...] * (jnp.exp(-norm) - means[...]) ** 2)
    xf = rbf * h_in
    pre1 = (hs_ @ w1hs[...] + hr_ @ w1hr[...] + xf @ w1x[...]
            + norm * w1n[...] + b1[...])
    mtx = _silu(pre1) @ w2[...] + b2[...]
    att = mtx @ watt[...] + batt[...]
    att = jnp.where(att > 0, att, 2.0 * (jnp.exp(att * 0.5) - 1.0))
    eatt = jnp.exp(att)
    b = hs_.shape[0]
    mtx_o[...] = mtx
    a16_o[...] = jnp.concatenate([eatt, jnp.zeros((b, 120), F32)], axis=1)
    xu = dx / (norm + 1e-5)
    xu8_o[...] = jnp.concatenate([xu, jnp.zeros((b, 5), F32)], axis=1)


def _edge_b_body(mtx, a16, xu8, dct, wx, rmat, tmat, wv, p_o):
    m = mtx[...]
    catt = a16[:, 0:4] / dct[:, 0:4]
    hea = (m @ rmat[...]) * (catt @ tmat[...])
    coeff = jnp.tanh(hea @ wx[...])
    s = (coeff @ wv[...])[:, 0:1]
    xu = xu8[:, 0:3]
    b = m.shape[0]
    p_o[...] = jnp.concatenate(
        [hea, xu[:, 0:1] * coeff, xu[:, 1:2] * coeff, xu[:, 2:3] * coeff,
         xu * s, jnp.zeros((b, PW - 1027), F32)], axis=1)


def _node_body(h, x, v, og, dcnt, wpn1, bpn1, wpn2, bpn2,
               wn1a, wn1b, wn1c, bn1, wn2, bn2, wvm1, bvm1, wvm2,
               h_o, x_o, v_o):
    h_, x_, v_ = h[...], x[...], v[...]
    og_ = og[...]
    he = og_[:, 0:256]
    cnt = dcnt[:, 4:5]
    cntc = jnp.maximum(cnt, 1.0)
    s0 = og_[:, 256:512]
    s1 = og_[:, 512:768]
    s2 = og_[:, 768:1024]
    comb_norm = (s0 * s0 + s1 * s1 + s2 * s2) / (cntc * cntc)
    t = _silu(comb_norm @ wpn1[...] + bpn1[...])
    hc = _silu(t @ wpn2[...] + bpn2[...])
    pre = h_ @ wn1a[...] + he @ wn1b[...] + hc @ wn1c[...] + bn1[...]
    out = _silu(_silu(pre) @ wn2[...] + bn2[...])
    h_new = h_ + out
    dv = og_[:, 1024:1027] / (cnt + 1e-10)
    t = _silu(h_new @ wvm1[...] + bvm1[...]) @ wvm2[...]
    scale = 2.0 * jax.nn.sigmoid(t[:, 0:1])
    v_new = dv + scale * v_
    h_o[...] = h_new
    x_o[...] = x_ + v_new
    v_o[...] = v_new


def _full(a):
    return pl.BlockSpec(a.shape, lambda i: (0,) * a.ndim)


def _tc_edge_a(hs, hr, xs, xr, ws):
    BE = 1280
    g = E // BE
    eb = lambda d: pl.BlockSpec((BE, d), lambda i: (i, 0))
    return pl.pallas_call(
        _edge_a_body,
        grid=(g,),
        in_specs=[eb(128), eb(128), eb(128), eb(128)] + [_full(w) for w in ws],
        out_specs=[eb(64), eb(128), eb(8)],
        out_shape=[jax.ShapeDtypeStruct((E, 64), F32),
                   jax.ShapeDtypeStruct((E, 128), F32),
                   jax.ShapeDtypeStruct((E, 8), F32)],
    )(hs, hr, xs, xr, *ws)


def _tc_edge_b(mtx, a16, xu8, dct, ws):
    BE = 640
    g = E // BE
    eb = lambda d: pl.BlockSpec((BE, d), lambda i: (i, 0))
    return pl.pallas_call(
        _edge_b_body,
        grid=(g,),
        in_specs=[eb(64), eb(128), eb(8), eb(128)] + [_full(w) for w in ws],
        out_specs=eb(PW),
        out_shape=jax.ShapeDtypeStruct((E, PW), F32),
    )(mtx, a16, xu8, dct, *ws)


def _tc_node(h, x, v, og, dcnt, ws):
    BN = 400
    g = N // BN
    nb = lambda d: pl.BlockSpec((BN, d), lambda i: (i, 0))
    return pl.pallas_call(
        _node_body,
        grid=(g,),
        in_specs=[nb(128), nb(3), nb(3), nb(PW), nb(128)]
        + [_full(w) for w in ws],
        out_specs=[nb(128), nb(3), nb(3)],
        out_shape=[jax.ShapeDtypeStruct((N, 128), F32),
                   jax.ShapeDtypeStruct((N, 3), F32),
                   jax.ShapeDtypeStruct((N, 3), F32)],
    )(h, x, v, og, dcnt, *ws)


def kernel(h, x, v, params, senders, receivers):
    p = params
    si = senders.astype(jnp.int32)
    ri = receivers.astype(jnp.int32)
    xtab = jnp.pad(x, ((0, 0), (0, 125)))
    zeros = jnp.zeros((N, 128), F32)

    padc = lambda w, n: jnp.pad(w, ((0, 0), (0, n)))
    row = lambda b: b.reshape(1, -1)

    wa = [
        padc(p['W_in'][:128], 14), padc(p['W_in'][128:], 14),
        row(jnp.pad(p['b_in'], (0, 14))),
        row(jnp.pad(p['betas'], (0, 14))), row(jnp.pad(p['means'], (0, 14))),
        p['W_eo1'][0:128], p['W_eo1'][128:256],
        jnp.pad(p['W_eo1'][256:306], ((0, 14), (0, 0))), p['W_eo1'][306:307],
        row(p['b_eo1']), p['W_eo2'], row(p['b_eo2']),
        padc(p['W_att'], 4), row(jnp.pad(p['b_att'], (0, 4))),
    ]
    wb = [
        p['W_xmix'],
        jnp.repeat(jnp.eye(64, dtype=F32), 4, axis=1),
        jnp.tile(jnp.eye(4, dtype=F32), (1, 64)),
        padc(p['W_vmix'], 7),
    ]
    wn = [
        p['W_pn1'], row(p['b_pn1']), p['W_pn2'], row(p['b_pn2']),
        p['W_n1'][0:128], p['W_n1'][128:384], p['W_n1'][384:448],
        row(p['b_n1']), p['W_n2'], row(p['b_n2']),
        p['W_vm1'], row(p['b_vm1']), padc(p['W_vm2'], 7),
    ]

    hs, hr, xs, xr = _sc_gather_nodes(h, xtab, si, ri)
    mtx, a16, xu8 = _tc_edge_a(hs, hr, xs, xr, wa)
    dcnt = _sc_scatter_dcnt(a16, ri, zeros)
    dct = _sc_gather_dcnt(dcnt, ri)
    pay = _tc_edge_b(mtx, a16, xu8, dct, wb)
    og = _sc_scatter_big(pay, ri, zeros)
    return _tc_node(h, x, v, og, dcnt, wn)
